# Initial kernel scaffold; baseline (speedup 1.0000x reference)
import jax
import jax.numpy as jnp
from jax.experimental import pallas as pl


def kernel(h, mem):
    raise NotImplementedError("write your pallas kernel here")


# speedup vs baseline: 1.0039x; 1.0039x over previous
import jax
import jax.numpy as jnp
from jax.experimental import pallas as pl

MAX_CTX = 32768
DIM = 2048

_NBLK = 32
_ROWS = MAX_CTX // _NBLK
_HALF = _NBLK // 2


def _copy_body(mem_ref, h_ref, out_ref):
    i = pl.program_id(0)

    @pl.when(i < _HALF)
    def _():
        out_ref[...] = mem_ref[...]

    @pl.when(i >= _HALF)
    def _():
        out_ref[...] = h_ref[...]


def kernel(h, mem):
    B, L, D = h.shape
    flat = h.reshape(B * L, D)
    new_mem = pl.pallas_call(
        _copy_body,
        grid=(_NBLK,),
        in_specs=[
            pl.BlockSpec((_ROWS, D),
                         lambda i: (jnp.where(i < _HALF, i + _HALF, _NBLK - 1), 0)),
            pl.BlockSpec((_ROWS, D),
                         lambda i: (jnp.where(i < _HALF, 0, i - _HALF), 0)),
        ],
        out_specs=pl.BlockSpec((_ROWS, D), lambda i: (i, 0)),
        out_shape=jax.ShapeDtypeStruct((MAX_CTX, D), h.dtype),
    )(mem, flat)
    return h, new_mem
